# baseline (speedup 1.0000x reference)
import jax
import jax.numpy as jnp
from jax import lax
from jax.experimental import pallas as pl
from jax.experimental.pallas import tpu as pltpu
from jax.experimental.pallas import tpu_sc as plsc

N_NODES = 10000
NP = 10240
IN_C = 128
HID_C = 64
OUT_C = 64
HEADS = 8
NEG_SLOPE = 0.2
E_RAW = 320000
E_TOT = E_RAW + N_NODES
EP = 335872
EROWS = EP // 128
ROWS_L1 = EROWS // 16
ROWS_L2 = EROWS // 32
NB = NP // 1024
HC2 = OUT_C // 2

_mesh = plsc.VectorSubcoreMesh(
    core_axis_name="c", subcore_axis_name="s", num_cores=2, num_subcores=16)
_sc_params = pltpu.CompilerParams(use_tc_tiling_on_sc=False,
                                  needs_layout_passes=False)


def _tcA_body(x_ref, w_ref, as_w_ref, ad_w_ref,
              h_ref, s_ref, d_ref, ms_ref, md_ref, sh_ref):
    i = pl.program_id(0)

    @pl.when(i == 0)
    def _init():
        ms_ref[...] = jnp.full((HEADS, 128), -jnp.inf, jnp.float32)
        md_ref[...] = jnp.full((HEADS, 128), -jnp.inf, jnp.float32)

    hblk = jnp.dot(x_ref[...], w_ref[...], preferred_element_type=jnp.float32)
    for h in range(HEADS):
        hh = hblk[:, h * HID_C:(h + 1) * HID_C]
        h_ref[h, 0] = hh[:, :HC2]
        h_ref[h, 1] = hh[:, HC2:]
        s = jnp.sum(hh * as_w_ref[h][None, :], axis=1)
        d = jnp.sum(hh * ad_w_ref[h][None, :], axis=1)
        s_ref[h] = s
        d_ref[h] = d
        ms_ref[h] = jnp.maximum(ms_ref[h], jnp.full((128,), jnp.max(s)))
        md_ref[h] = jnp.maximum(md_ref[h], jnp.full((128,), jnp.max(d)))

    @pl.when(i == NB - 1)
    def _fin():
        t = ms_ref[...] + md_ref[...]
        t = jnp.where(t > 0, t, NEG_SLOPE * t)
        sh_ref[...] = jnp.maximum(t, 0.0)


def _tcA(xp, W1, as_w, ad_w):
    f32 = jnp.float32
    return pl.pallas_call(
        _tcA_body,
        grid=(NB,),
        in_specs=[
            pl.BlockSpec((1024, IN_C), lambda i: (i, 0)),
            pl.BlockSpec((IN_C, HEADS * HID_C), lambda i: (0, 0)),
            pl.BlockSpec((HEADS, HID_C), lambda i: (0, 0)),
            pl.BlockSpec((HEADS, HID_C), lambda i: (0, 0)),
        ],
        out_specs=[
            pl.BlockSpec((HEADS, 2, 1024, HC2), lambda i: (0, 0, i, 0)),
            pl.BlockSpec((HEADS, 1024), lambda i: (0, i)),
            pl.BlockSpec((HEADS, 1024), lambda i: (0, i)),
            pl.BlockSpec((HEADS, 128), lambda i: (0, 0)),
            pl.BlockSpec((HEADS, 128), lambda i: (0, 0)),
            pl.BlockSpec((HEADS, 128), lambda i: (0, 0)),
        ],
        out_shape=[
            jax.ShapeDtypeStruct((HEADS, 2, NP, HC2), f32),
            jax.ShapeDtypeStruct((HEADS, NP), f32),
            jax.ShapeDtypeStruct((HEADS, NP), f32),
            jax.ShapeDtypeStruct((HEADS, 128), f32),
            jax.ShapeDtypeStruct((HEADS, 128), f32),
            jax.ShapeDtypeStruct((HEADS, 128), f32),
        ],
    )(xp, W1, as_w, ad_w)


def _edge_pass(rows, cw, hoff, vm_src, vm_dst, vm_as, vm_ad, vm_sh, hsrc,
               spm_o, spm_d, sets, den_pred):
    iota = lax.iota(jnp.int32, 16)

    def _eix(j, S):
        h_, m_, e_, es_, ix_, gs_, ms_, ds_ = S
        for k in range(8):
            s16 = vm_src[j, pl.ds(k * 16, 16)]
            d16 = vm_dst[j, pl.ds(k * 16, 16)]
            ix_[pl.ds(k * 16, 16)] = s16 + hoff
            sv = plsc.load_gather(vm_as, [s16])
            dv = plsc.load_gather(vm_ad, [d16])
            a = sv + dv
            a = jnp.where(a > 0, a, NEG_SLOPE * a)
            e_[pl.ds(k * 16, 16)] = jnp.exp(a - vm_sh[pl.ds(0, 16)])
        pltpu.async_copy(hsrc.at[ix_], h_, gs_)

    def _half(j, t, n_t, S):
        h_, m_, e_, es_, ix_, gs_, ms_, ds_ = S

        @pl.when(t > 0)
        def _w():
            pltpu.make_async_copy(m_, spm_o.at[vm_dst.at[j]], ms_).wait()

        @pl.when(jnp.logical_and(t > 0, den_pred))
        def _wd():
            pltpu.make_async_copy(es_, spm_d.at[vm_dst.at[j]], ds_).wait()

        pltpu.make_async_copy(hsrc.at[ix_], h_, gs_).wait()

        ev = [e_[pl.ds(g * 16, 16)] for g in range(8)]
        rows16 = [iota + (g * 16) for g in range(8)]
        for c in range(cw):
            c16 = jnp.full((16,), c, jnp.int32)
            for g in range(8):
                hv = plsc.load_gather(h_, [rows16[g], c16])
                plsc.store_scatter(m_, [rows16[g], c16], hv * ev[g])
        pltpu.async_copy(m_, spm_o.at[vm_dst.at[j]], ms_, add=True)

        @pl.when(den_pred)
        def _sd():
            for k in range(8):
                es_[pl.ds(k * 16, 16)] = e_[pl.ds(k * 16, 16)]
            pltpu.async_copy(es_, spm_d.at[vm_dst.at[j]], ds_, add=True)

        @pl.when(t < n_t - 1)
        def _p():
            _eix(j + 2, S)

    S0, S1 = sets
    n_t = rows // 2
    _eix(0, S0)
    _eix(1, S1)

    def _body(t, c):
        _half(2 * t, t, n_t, S0)
        _half(2 * t + 1, t, n_t, S1)
        return c

    lax.fori_loop(0, n_t, _body, 0)
    for S in sets:
        h_, m_, e_, es_, ix_, gs_, ms_, ds_ = S
        pltpu.make_async_copy(m_, spm_o.at[vm_dst.at[0]], ms_).wait()

        @pl.when(den_pred)
        def _wd2():
            pltpu.make_async_copy(es_, spm_d.at[vm_dst.at[0]], ds_).wait()


def _zero_bufs(zb, zd, cq=4):
    def _zrow(r, c):
        for q in range(cq):
            zb[r, pl.ds(q * 16, 16)] = jnp.zeros((16,), jnp.float32)
        return c

    lax.fori_loop(0, 128, _zrow, 0)

    def _zdrow(r, c):
        zd[pl.ds(r * 16, 16)] = jnp.zeros((16,), jnp.float32)
        return c

    lax.fori_loop(0, 40, _zdrow, 0)


def _sc1_body(srcr, dstr, asp, adp, shp, h1f,
              out_o, out_d,
              vm_src, vm_dst, vm_as, vm_ad, vm_sh,
              h0, m0, e0, es0, ix0, h1, m1, e1, es1, ix1,
              zb, zd, spm_o, spm_d, gs0, ms0, ds0, gs1, ms1, ds1):
    core = lax.axis_index("c")
    sub = lax.axis_index("s")
    w0 = sub * 640
    sets = ((h0, m0, e0, es0, ix0, gs0, ms0, ds0),
            (h1, m1, e1, es1, ix1, gs1, ms1, ds1))

    _zero_bufs(zb, zd, cq=HC2 // 16)
    pltpu.sync_copy(srcr.at[sub], vm_src)
    pltpu.sync_copy(dstr.at[sub], vm_dst)

    def _pass(p, c):
        head = core * 4 + p // 2
        den = (p % 2) == 0
        aoff = pl.multiple_of(head * NP, 128)
        hoff = pl.multiple_of((core * 8 + p) * NP, 128)

        @pl.when(den)
        def _cp():
            pltpu.sync_copy(asp.at[pl.ds(aoff, NP)], vm_as)
            pltpu.sync_copy(adp.at[pl.ds(aoff, NP)], vm_ad)
            pltpu.sync_copy(
                shp.at[pl.ds(pl.multiple_of(head * 128, 128), 128)], vm_sh)
            pltpu.sync_copy(zd, spm_d.at[pl.ds(w0, 640)])

        for b in range(5):
            pltpu.sync_copy(zb, spm_o.at[pl.ds(w0 + b * 128, 128)])
        plsc.subcore_barrier()
        _edge_pass(ROWS_L1, HC2, hoff, vm_src, vm_dst, vm_as,
                   vm_ad, vm_sh, h1f, spm_o, spm_d, sets, den)
        plsc.subcore_barrier()
        pltpu.sync_copy(spm_o.at[pl.ds(w0, 640)],
                        out_o.at[core * 8 + p, pl.ds(w0, 640)])

        @pl.when(den)
        def _dd():
            pltpu.sync_copy(
                spm_d.at[pl.ds(w0, 640)],
                out_d.at[pl.ds(pl.multiple_of(aoff + w0, 128), 640)])

        plsc.subcore_barrier()
        return c

    lax.fori_loop(0, 8, _pass, 0)


def _sc_edges1(srcr, dstr, asp, adp, shp, h1f):
    f32 = jnp.float32
    i32 = jnp.int32
    bufset = [
        pltpu.VMEM((128, HC2), f32),
        pltpu.VMEM((128, HC2), f32),
        pltpu.VMEM((128,), f32),
        pltpu.VMEM((128,), f32),
        pltpu.VMEM((128,), i32),
    ]
    fn = pl.kernel(
        _sc1_body,
        out_type=[
            jax.ShapeDtypeStruct((HEADS * 2, NP, HC2), f32),
            jax.ShapeDtypeStruct((HEADS * NP,), f32),
        ],
        mesh=_mesh,
        compiler_params=_sc_params,
        scratch_types=[
            pltpu.VMEM((ROWS_L1, 128), i32),
            pltpu.VMEM((ROWS_L1, 128), i32),
            pltpu.VMEM((NP,), f32),
            pltpu.VMEM((NP,), f32),
            pltpu.VMEM((128,), f32),
        ] + bufset + bufset + [
            pltpu.VMEM((128, HC2), f32),
            pltpu.VMEM((640,), f32),
            pltpu.VMEM_SHARED((NP, HC2), f32),
            pltpu.VMEM_SHARED((NP,), f32),
        ] + [pltpu.SemaphoreType.DMA] * 6,
    )
    return fn(srcr, dstr, asp, adp, shp, h1f)


def _tcC_body(p_ref, d_ref, b1_ref, w2_ref, as2_w_ref, ad2_w_ref,
              h2_ref, s2_ref, d2_ref, ms_ref, md_ref, sh_ref):
    i = pl.program_id(0)

    @pl.when(i == 0)
    def _init():
        ms_ref[...] = jnp.full((128,), -jnp.inf, jnp.float32)
        md_ref[...] = jnp.full((128,), -jnp.inf, jnp.float32)

    acc = jnp.zeros((1024, OUT_C), jnp.float32)
    for h in range(HEADS):
        ph = jnp.concatenate((p_ref[h, 0], p_ref[h, 1]), axis=1)
        v = ph / (d_ref[h][:, None] + 1e-16) + b1_ref[h][None, :]
        v = jnp.where(v > 0, v, jnp.exp(v) - 1.0)
        acc = acc + jnp.dot(v, w2_ref[h], preferred_element_type=jnp.float32)
    h2_ref[0] = acc[:, :32]
    h2_ref[1] = acc[:, 32:]
    s2 = jnp.sum(acc * as2_w_ref[0][None, :], axis=1)
    d2 = jnp.sum(acc * ad2_w_ref[0][None, :], axis=1)
    s2_ref[...] = s2
    d2_ref[...] = d2
    ms_ref[...] = jnp.maximum(ms_ref[...], jnp.full((128,), jnp.max(s2)))
    md_ref[...] = jnp.maximum(md_ref[...], jnp.full((128,), jnp.max(d2)))

    @pl.when(i == NB - 1)
    def _fin():
        t = ms_ref[...] + md_ref[...]
        t = jnp.where(t > 0, t, NEG_SLOPE * t)
        sh_ref[...] = jnp.maximum(t, 0.0)


def _tcC(out1, den1, b1r, w2r, as2_w, ad2_w):
    f32 = jnp.float32
    return pl.pallas_call(
        _tcC_body,
        grid=(NB,),
        in_specs=[
            pl.BlockSpec((HEADS, 2, 1024, HC2), lambda i: (0, 0, i, 0)),
            pl.BlockSpec((HEADS, 1024), lambda i: (0, i)),
            pl.BlockSpec((HEADS, HID_C), lambda i: (0, 0)),
            pl.BlockSpec((HEADS, HID_C, OUT_C), lambda i: (0, 0, 0)),
            pl.BlockSpec((1, OUT_C), lambda i: (0, 0)),
            pl.BlockSpec((1, OUT_C), lambda i: (0, 0)),
        ],
        out_specs=[
            pl.BlockSpec((2, 1024, OUT_C // 2), lambda i: (0, i, 0)),
            pl.BlockSpec((1024,), lambda i: (i,)),
            pl.BlockSpec((1024,), lambda i: (i,)),
            pl.BlockSpec((128,), lambda i: (0,)),
            pl.BlockSpec((128,), lambda i: (0,)),
            pl.BlockSpec((128,), lambda i: (0,)),
        ],
        out_shape=[
            jax.ShapeDtypeStruct((2, NP, OUT_C // 2), f32),
            jax.ShapeDtypeStruct((NP,), f32),
            jax.ShapeDtypeStruct((NP,), f32),
            jax.ShapeDtypeStruct((128,), f32),
            jax.ShapeDtypeStruct((128,), f32),
            jax.ShapeDtypeStruct((128,), f32),
        ],
    )(out1, den1, b1r, w2r, as2_w, ad2_w)


def _sc2_body(srcr, dstr, asp, adp, shp, h2f,
              out_o, out_d,
              vm_src, vm_dst, vm_as, vm_ad, vm_sh,
              h0, m0, e0, es0, ix0, h1, m1, e1, es1, ix1,
              zb, zd, spm_o, spm_d, gs0, ms0, ds0, gs1, ms1, ds1):
    core = lax.axis_index("c")
    sub = lax.axis_index("s")
    w0 = sub * 640
    sets = ((h0, m0, e0, es0, ix0, gs0, ms0, ds0),
            (h1, m1, e1, es1, ix1, gs1, ms1, ds1))

    _zero_bufs(zb, zd, cq=HC2 // 16)
    pltpu.sync_copy(srcr.at[sub], vm_src)
    pltpu.sync_copy(dstr.at[sub], vm_dst)
    pltpu.sync_copy(asp, vm_as)
    pltpu.sync_copy(adp, vm_ad)
    pltpu.sync_copy(shp, vm_sh)
    for b in range(5):
        pltpu.sync_copy(zb, spm_o.at[pl.ds(w0 + b * 128, 128)])
    pltpu.sync_copy(zd, spm_d.at[pl.ds(w0, 640)])
    plsc.subcore_barrier()
    hoff = pl.multiple_of(core * NP, 128)
    _edge_pass(ROWS_L1, HC2, hoff, vm_src, vm_dst, vm_as, vm_ad,
               vm_sh, h2f, spm_o, spm_d, sets, core == 0)
    plsc.subcore_barrier()
    pltpu.sync_copy(spm_o.at[pl.ds(w0, 640)], out_o.at[core, pl.ds(w0, 640)])

    @pl.when(core == 0)
    def _dd():
        pltpu.sync_copy(spm_d.at[pl.ds(w0, 640)], out_d.at[pl.ds(w0, 640)])


def _sc_edges2(srcr, dstr, asp, adp, shp, h2f):
    f32 = jnp.float32
    i32 = jnp.int32
    bufset = [
        pltpu.VMEM((128, HC2), f32),
        pltpu.VMEM((128, HC2), f32),
        pltpu.VMEM((128,), f32),
        pltpu.VMEM((128,), f32),
        pltpu.VMEM((128,), i32),
    ]
    fn = pl.kernel(
        _sc2_body,
        out_type=[
            jax.ShapeDtypeStruct((2, NP, HC2), f32),
            jax.ShapeDtypeStruct((NP,), f32),
        ],
        mesh=_mesh,
        compiler_params=_sc_params,
        scratch_types=[
            pltpu.VMEM((ROWS_L1, 128), i32),
            pltpu.VMEM((ROWS_L1, 128), i32),
            pltpu.VMEM((NP,), f32),
            pltpu.VMEM((NP,), f32),
            pltpu.VMEM((128,), f32),
        ] + bufset + bufset + [
            pltpu.VMEM((128, HC2), f32),
            pltpu.VMEM((640,), f32),
            pltpu.VMEM_SHARED((NP, HC2), f32),
            pltpu.VMEM_SHARED((NP,), f32),
        ] + [pltpu.SemaphoreType.DMA] * 6,
    )
    return fn(srcr, dstr, asp, adp, shp, h2f)


def _tcE_body(p_ref, d_ref, b2_ref, o_ref):
    den = d_ref[...]
    full = jnp.concatenate((p_ref[0], p_ref[1]), axis=1)
    o_ref[...] = full / (den[:, None] + 1e-16) + b2_ref[0][None, :]


def _tcE(out2, den2, b2r):
    return pl.pallas_call(
        _tcE_body,
        grid=(NB,),
        in_specs=[
            pl.BlockSpec((2, 1024, HC2), lambda i: (0, i, 0)),
            pl.BlockSpec((1024,), lambda i: (i,)),
            pl.BlockSpec((1, OUT_C), lambda i: (0, 0)),
        ],
        out_specs=pl.BlockSpec((1024, OUT_C), lambda i: (i, 0)),
        out_shape=jax.ShapeDtypeStruct((NP, OUT_C), jnp.float32),
    )(out2, den2, b2r)


def kernel(x, edge_index, W1, att_src1, att_dst1, b1, W2, att_src2,
           att_dst2, b2):
    n = x.shape[0]
    i32 = jnp.int32
    loop = jnp.arange(n, dtype=i32)
    pad = jnp.full((EP - E_TOT,), NP - 1, i32)
    src = jnp.concatenate([edge_index[0].astype(i32), loop, pad])
    dst = jnp.concatenate([edge_index[1].astype(i32), loop, pad])
    srcr16 = src.reshape(16, ROWS_L1, 128)
    dstr16 = dst.reshape(16, ROWS_L1, 128)
    xp = jnp.pad(x, ((0, NP - n), (0, 0)))

    h1p, asp, adp, _, _, sh1 = _tcA(xp, W1, att_src1, att_dst1)
    out1, den1 = _sc_edges1(srcr16, dstr16, asp.reshape(HEADS * NP),
                            adp.reshape(HEADS * NP),
                            sh1.reshape(HEADS * 128),
                            h1p.reshape(HEADS * 2 * NP, HC2))
    h2s, as2, ad2, _, _, sh2 = _tcC(out1.reshape(HEADS, 2, NP, HC2),
                                    den1.reshape(HEADS, NP),
                                    b1.reshape(HEADS, HID_C),
                                    W2.reshape(HEADS, HID_C, OUT_C),
                                    att_src2, att_dst2)
    out2, den2 = _sc_edges2(srcr16, dstr16, as2, ad2, sh2,
                            h2s.reshape(2 * NP, HC2))
    out = _tcE(out2, den2, b2.reshape(1, OUT_C))
    return out[:n]


# speedup vs baseline: 2.7929x; 2.7929x over previous
import jax
import jax.numpy as jnp
from jax import lax
from jax.experimental import pallas as pl
from jax.experimental.pallas import tpu as pltpu
from jax.experimental.pallas import tpu_sc as plsc

N_NODES = 10000
NP = 10240
IN_C = 128
HID_C = 64
OUT_C = 64
HEADS = 8
NEG_SLOPE = 0.2
E_RAW = 320000
E_TOT = E_RAW + N_NODES
EP = 335872
EROWS = EP // 128
ROWS_L1 = EROWS // 16
ROWS_L2 = EROWS // 32
NB = NP // 1024
HC2 = OUT_C // 2

_mesh = plsc.VectorSubcoreMesh(
    core_axis_name="c", subcore_axis_name="s", num_cores=2, num_subcores=16)
_sc_params = pltpu.CompilerParams(use_tc_tiling_on_sc=False,
                                  needs_layout_passes=False)


def _tcA_body(x_ref, w_ref, as_w_ref, ad_w_ref,
              h_ref, s_ref, d_ref, ms_ref, md_ref, sh_ref):
    i = pl.program_id(0)

    @pl.when(i == 0)
    def _init():
        ms_ref[...] = jnp.full((HEADS, 128), -jnp.inf, jnp.float32)
        md_ref[...] = jnp.full((HEADS, 128), -jnp.inf, jnp.float32)

    hblk = jnp.dot(x_ref[...], w_ref[...], preferred_element_type=jnp.float32)
    for h in range(HEADS):
        hh = hblk[:, h * HID_C:(h + 1) * HID_C]
        h_ref[h, 0] = hh[:, :HC2]
        h_ref[h, 1] = hh[:, HC2:]
        s = jnp.sum(hh * as_w_ref[h][None, :], axis=1)
        d = jnp.sum(hh * ad_w_ref[h][None, :], axis=1)
        s_ref[h] = s
        d_ref[h] = d
        ms_ref[h] = jnp.maximum(ms_ref[h], jnp.full((128,), jnp.max(s)))
        md_ref[h] = jnp.maximum(md_ref[h], jnp.full((128,), jnp.max(d)))

    @pl.when(i == NB - 1)
    def _fin():
        t = ms_ref[...] + md_ref[...]
        t = jnp.where(t > 0, t, NEG_SLOPE * t)
        sh_ref[...] = jnp.maximum(t, 0.0)


def _tcA(xp, W1, as_w, ad_w):
    f32 = jnp.float32
    return pl.pallas_call(
        _tcA_body,
        grid=(NB,),
        in_specs=[
            pl.BlockSpec((1024, IN_C), lambda i: (i, 0)),
            pl.BlockSpec((IN_C, HEADS * HID_C), lambda i: (0, 0)),
            pl.BlockSpec((HEADS, HID_C), lambda i: (0, 0)),
            pl.BlockSpec((HEADS, HID_C), lambda i: (0, 0)),
        ],
        out_specs=[
            pl.BlockSpec((HEADS, 2, 1024, HC2), lambda i: (0, 0, i, 0)),
            pl.BlockSpec((HEADS, 1024), lambda i: (0, i)),
            pl.BlockSpec((HEADS, 1024), lambda i: (0, i)),
            pl.BlockSpec((HEADS, 128), lambda i: (0, 0)),
            pl.BlockSpec((HEADS, 128), lambda i: (0, 0)),
            pl.BlockSpec((HEADS, 128), lambda i: (0, 0)),
        ],
        out_shape=[
            jax.ShapeDtypeStruct((HEADS, 2, NP, HC2), f32),
            jax.ShapeDtypeStruct((HEADS, NP), f32),
            jax.ShapeDtypeStruct((HEADS, NP), f32),
            jax.ShapeDtypeStruct((HEADS, 128), f32),
            jax.ShapeDtypeStruct((HEADS, 128), f32),
            jax.ShapeDtypeStruct((HEADS, 128), f32),
        ],
    )(xp, W1, as_w, ad_w)


def _edge_pass(rows, cw, hoff, vm_src, vm_dst, vm_as, vm_ad, vm_sh, hsrc,
               spm_o, spm_d, sets, den_pred):
    iota = lax.iota(jnp.int32, 16)

    def _eix(j, S):
        h_, m_, e_, es_, ix_, gs_, ms_, ds_ = S
        for k in range(8):
            s16 = vm_src[j, pl.ds(k * 16, 16)]
            d16 = vm_dst[j, pl.ds(k * 16, 16)]
            ix_[pl.ds(k * 16, 16)] = s16 + hoff
            sv = plsc.load_gather(vm_as, [s16])
            dv = plsc.load_gather(vm_ad, [d16])
            a = sv + dv
            a = jnp.where(a > 0, a, NEG_SLOPE * a)
            e_[pl.ds(k * 16, 16)] = jnp.exp(a - vm_sh[pl.ds(0, 16)])
        pltpu.async_copy(hsrc.at[ix_], h_, gs_)

    def _half(j, t, n_t, S):
        h_, m_, e_, es_, ix_, gs_, ms_, ds_ = S

        @pl.when(t > 0)
        def _w():
            pltpu.make_async_copy(m_, spm_o.at[vm_dst.at[j]], ms_).wait()

        @pl.when(jnp.logical_and(t > 0, den_pred))
        def _wd():
            pltpu.make_async_copy(es_, spm_d.at[vm_dst.at[j]], ds_).wait()

        pltpu.make_async_copy(hsrc.at[ix_], h_, gs_).wait()

        def _scale(r, c):
            ev = plsc.load_gather(e_, [jnp.full((16,), r, jnp.int32)])
            for q in range(cw // 16):
                m_[r, pl.ds(q * 16, 16)] = h_[r, pl.ds(q * 16, 16)] * ev
            return c

        lax.fori_loop(0, 128, _scale, 0)
        pltpu.async_copy(m_, spm_o.at[vm_dst.at[j]], ms_, add=True)

        @pl.when(den_pred)
        def _sd():
            for k in range(8):
                es_[pl.ds(k * 16, 16)] = e_[pl.ds(k * 16, 16)]
            pltpu.async_copy(es_, spm_d.at[vm_dst.at[j]], ds_, add=True)

        @pl.when(t < n_t - 1)
        def _p():
            _eix(j + 2, S)

    S0, S1 = sets
    n_t = rows // 2
    _eix(0, S0)
    _eix(1, S1)

    def _body(t, c):
        _half(2 * t, t, n_t, S0)
        _half(2 * t + 1, t, n_t, S1)
        return c

    lax.fori_loop(0, n_t, _body, 0)
    for S in sets:
        h_, m_, e_, es_, ix_, gs_, ms_, ds_ = S
        pltpu.make_async_copy(m_, spm_o.at[vm_dst.at[0]], ms_).wait()

        @pl.when(den_pred)
        def _wd2():
            pltpu.make_async_copy(es_, spm_d.at[vm_dst.at[0]], ds_).wait()


def _zero_bufs(zb, zd, cq=4):
    def _zrow(r, c):
        for q in range(cq):
            zb[r, pl.ds(q * 16, 16)] = jnp.zeros((16,), jnp.float32)
        return c

    lax.fori_loop(0, 128, _zrow, 0)

    def _zdrow(r, c):
        zd[pl.ds(r * 16, 16)] = jnp.zeros((16,), jnp.float32)
        return c

    lax.fori_loop(0, 40, _zdrow, 0)


def _sc1_body(srcr, dstr, asp, adp, shp, h1f,
              out_o, out_d,
              vm_src, vm_dst, vm_as, vm_ad, vm_sh,
              h0, m0, e0, es0, ix0, h1, m1, e1, es1, ix1,
              zb, zd, spm_o, spm_d, gs0, ms0, ds0, gs1, ms1, ds1):
    core = lax.axis_index("c")
    sub = lax.axis_index("s")
    w0 = sub * 640
    sets = ((h0, m0, e0, es0, ix0, gs0, ms0, ds0),
            (h1, m1, e1, es1, ix1, gs1, ms1, ds1))

    _zero_bufs(zb, zd, cq=HC2 // 16)
    pltpu.sync_copy(srcr.at[sub], vm_src)
    pltpu.sync_copy(dstr.at[sub], vm_dst)

    def _pass(p, c):
        head = core * 4 + p // 2
        den = (p % 2) == 0
        aoff = pl.multiple_of(head * NP, 128)
        hoff = pl.multiple_of((core * 8 + p) * NP, 128)

        @pl.when(den)
        def _cp():
            pltpu.sync_copy(asp.at[pl.ds(aoff, NP)], vm_as)
            pltpu.sync_copy(adp.at[pl.ds(aoff, NP)], vm_ad)
            pltpu.sync_copy(
                shp.at[pl.ds(pl.multiple_of(head * 128, 128), 128)], vm_sh)
            pltpu.sync_copy(zd, spm_d.at[pl.ds(w0, 640)])

        for b in range(5):
            pltpu.sync_copy(zb, spm_o.at[pl.ds(w0 + b * 128, 128)])
        plsc.subcore_barrier()
        _edge_pass(ROWS_L1, HC2, hoff, vm_src, vm_dst, vm_as,
                   vm_ad, vm_sh, h1f, spm_o, spm_d, sets, den)
        plsc.subcore_barrier()
        pltpu.sync_copy(spm_o.at[pl.ds(w0, 640)],
                        out_o.at[core * 8 + p, pl.ds(w0, 640)])

        @pl.when(den)
        def _dd():
            pltpu.sync_copy(
                spm_d.at[pl.ds(w0, 640)],
                out_d.at[pl.ds(pl.multiple_of(aoff + w0, 128), 640)])

        plsc.subcore_barrier()
        return c

    lax.fori_loop(0, 8, _pass, 0)


def _sc_edges1(srcr, dstr, asp, adp, shp, h1f):
    f32 = jnp.float32
    i32 = jnp.int32
    bufset = [
        pltpu.VMEM((128, HC2), f32),
        pltpu.VMEM((128, HC2), f32),
        pltpu.VMEM((128,), f32),
        pltpu.VMEM((128,), f32),
        pltpu.VMEM((128,), i32),
    ]
    fn = pl.kernel(
        _sc1_body,
        out_type=[
            jax.ShapeDtypeStruct((HEADS * 2, NP, HC2), f32),
            jax.ShapeDtypeStruct((HEADS * NP,), f32),
        ],
        mesh=_mesh,
        compiler_params=_sc_params,
        scratch_types=[
            pltpu.VMEM((ROWS_L1, 128), i32),
            pltpu.VMEM((ROWS_L1, 128), i32),
            pltpu.VMEM((NP,), f32),
            pltpu.VMEM((NP,), f32),
            pltpu.VMEM((128,), f32),
        ] + bufset + bufset + [
            pltpu.VMEM((128, HC2), f32),
            pltpu.VMEM((640,), f32),
            pltpu.VMEM_SHARED((NP, HC2), f32),
            pltpu.VMEM_SHARED((NP,), f32),
        ] + [pltpu.SemaphoreType.DMA] * 6,
    )
    return fn(srcr, dstr, asp, adp, shp, h1f)


def _tcC_body(p_ref, d_ref, b1_ref, w2_ref, as2_w_ref, ad2_w_ref,
              h2_ref, s2_ref, d2_ref, ms_ref, md_ref, sh_ref):
    i = pl.program_id(0)

    @pl.when(i == 0)
    def _init():
        ms_ref[...] = jnp.full((128,), -jnp.inf, jnp.float32)
        md_ref[...] = jnp.full((128,), -jnp.inf, jnp.float32)

    acc = jnp.zeros((1024, OUT_C), jnp.float32)
    for h in range(HEADS):
        ph = jnp.concatenate((p_ref[h, 0], p_ref[h, 1]), axis=1)
        v = ph / (d_ref[h][:, None] + 1e-16) + b1_ref[h][None, :]
        v = jnp.where(v > 0, v, jnp.exp(v) - 1.0)
        acc = acc + jnp.dot(v, w2_ref[h], preferred_element_type=jnp.float32)
    h2_ref[0] = acc[:, :32]
    h2_ref[1] = acc[:, 32:]
    s2 = jnp.sum(acc * as2_w_ref[0][None, :], axis=1)
    d2 = jnp.sum(acc * ad2_w_ref[0][None, :], axis=1)
    s2_ref[...] = s2
    d2_ref[...] = d2
    ms_ref[...] = jnp.maximum(ms_ref[...], jnp.full((128,), jnp.max(s2)))
    md_ref[...] = jnp.maximum(md_ref[...], jnp.full((128,), jnp.max(d2)))

    @pl.when(i == NB - 1)
    def _fin():
        t = ms_ref[...] + md_ref[...]
        t = jnp.where(t > 0, t, NEG_SLOPE * t)
        sh_ref[...] = jnp.maximum(t, 0.0)


def _tcC(out1, den1, b1r, w2r, as2_w, ad2_w):
    f32 = jnp.float32
    return pl.pallas_call(
        _tcC_body,
        grid=(NB,),
        in_specs=[
            pl.BlockSpec((HEADS, 2, 1024, HC2), lambda i: (0, 0, i, 0)),
            pl.BlockSpec((HEADS, 1024), lambda i: (0, i)),
            pl.BlockSpec((HEADS, HID_C), lambda i: (0, 0)),
            pl.BlockSpec((HEADS, HID_C, OUT_C), lambda i: (0, 0, 0)),
            pl.BlockSpec((1, OUT_C), lambda i: (0, 0)),
            pl.BlockSpec((1, OUT_C), lambda i: (0, 0)),
        ],
        out_specs=[
            pl.BlockSpec((2, 1024, OUT_C // 2), lambda i: (0, i, 0)),
            pl.BlockSpec((1024,), lambda i: (i,)),
            pl.BlockSpec((1024,), lambda i: (i,)),
            pl.BlockSpec((128,), lambda i: (0,)),
            pl.BlockSpec((128,), lambda i: (0,)),
            pl.BlockSpec((128,), lambda i: (0,)),
        ],
        out_shape=[
            jax.ShapeDtypeStruct((2, NP, OUT_C // 2), f32),
            jax.ShapeDtypeStruct((NP,), f32),
            jax.ShapeDtypeStruct((NP,), f32),
            jax.ShapeDtypeStruct((128,), f32),
            jax.ShapeDtypeStruct((128,), f32),
            jax.ShapeDtypeStruct((128,), f32),
        ],
    )(out1, den1, b1r, w2r, as2_w, ad2_w)


def _sc2_body(srcr, dstr, asp, adp, shp, h2f,
              out_o, out_d,
              vm_src, vm_dst, vm_as, vm_ad, vm_sh,
              h0, m0, e0, es0, ix0, h1, m1, e1, es1, ix1,
              zb, zd, spm_o, spm_d, gs0, ms0, ds0, gs1, ms1, ds1):
    core = lax.axis_index("c")
    sub = lax.axis_index("s")
    w0 = sub * 640
    sets = ((h0, m0, e0, es0, ix0, gs0, ms0, ds0),
            (h1, m1, e1, es1, ix1, gs1, ms1, ds1))

    _zero_bufs(zb, zd, cq=HC2 // 16)
    pltpu.sync_copy(srcr.at[sub], vm_src)
    pltpu.sync_copy(dstr.at[sub], vm_dst)
    pltpu.sync_copy(asp, vm_as)
    pltpu.sync_copy(adp, vm_ad)
    pltpu.sync_copy(shp, vm_sh)
    for b in range(5):
        pltpu.sync_copy(zb, spm_o.at[pl.ds(w0 + b * 128, 128)])
    pltpu.sync_copy(zd, spm_d.at[pl.ds(w0, 640)])
    plsc.subcore_barrier()
    hoff = pl.multiple_of(core * NP, 128)
    _edge_pass(ROWS_L1, HC2, hoff, vm_src, vm_dst, vm_as, vm_ad,
               vm_sh, h2f, spm_o, spm_d, sets, core == 0)
    plsc.subcore_barrier()
    pltpu.sync_copy(spm_o.at[pl.ds(w0, 640)], out_o.at[core, pl.ds(w0, 640)])

    @pl.when(core == 0)
    def _dd():
        pltpu.sync_copy(spm_d.at[pl.ds(w0, 640)], out_d.at[pl.ds(w0, 640)])


def _sc_edges2(srcr, dstr, asp, adp, shp, h2f):
    f32 = jnp.float32
    i32 = jnp.int32
    bufset = [
        pltpu.VMEM((128, HC2), f32),
        pltpu.VMEM((128, HC2), f32),
        pltpu.VMEM((128,), f32),
        pltpu.VMEM((128,), f32),
        pltpu.VMEM((128,), i32),
    ]
    fn = pl.kernel(
        _sc2_body,
        out_type=[
            jax.ShapeDtypeStruct((2, NP, HC2), f32),
            jax.ShapeDtypeStruct((NP,), f32),
        ],
        mesh=_mesh,
        compiler_params=_sc_params,
        scratch_types=[
            pltpu.VMEM((ROWS_L1, 128), i32),
            pltpu.VMEM((ROWS_L1, 128), i32),
            pltpu.VMEM((NP,), f32),
            pltpu.VMEM((NP,), f32),
            pltpu.VMEM((128,), f32),
        ] + bufset + bufset + [
            pltpu.VMEM((128, HC2), f32),
            pltpu.VMEM((640,), f32),
            pltpu.VMEM_SHARED((NP, HC2), f32),
            pltpu.VMEM_SHARED((NP,), f32),
        ] + [pltpu.SemaphoreType.DMA] * 6,
    )
    return fn(srcr, dstr, asp, adp, shp, h2f)


def _tcE_body(p_ref, d_ref, b2_ref, o_ref):
    den = d_ref[...]
    full = jnp.concatenate((p_ref[0], p_ref[1]), axis=1)
    o_ref[...] = full / (den[:, None] + 1e-16) + b2_ref[0][None, :]


def _tcE(out2, den2, b2r):
    return pl.pallas_call(
        _tcE_body,
        grid=(NB,),
        in_specs=[
            pl.BlockSpec((2, 1024, HC2), lambda i: (0, i, 0)),
            pl.BlockSpec((1024,), lambda i: (i,)),
            pl.BlockSpec((1, OUT_C), lambda i: (0, 0)),
        ],
        out_specs=pl.BlockSpec((1024, OUT_C), lambda i: (i, 0)),
        out_shape=jax.ShapeDtypeStruct((NP, OUT_C), jnp.float32),
    )(out2, den2, b2r)


def kernel(x, edge_index, W1, att_src1, att_dst1, b1, W2, att_src2,
           att_dst2, b2):
    n = x.shape[0]
    i32 = jnp.int32
    loop = jnp.arange(n, dtype=i32)
    pad = jnp.full((EP - E_TOT,), NP - 1, i32)
    src = jnp.concatenate([edge_index[0].astype(i32), loop, pad])
    dst = jnp.concatenate([edge_index[1].astype(i32), loop, pad])
    srcr16 = src.reshape(16, ROWS_L1, 128)
    dstr16 = dst.reshape(16, ROWS_L1, 128)
    xp = jnp.pad(x, ((0, NP - n), (0, 0)))

    h1p, asp, adp, _, _, sh1 = _tcA(xp, W1, att_src1, att_dst1)
    out1, den1 = _sc_edges1(srcr16, dstr16, asp.reshape(HEADS * NP),
                            adp.reshape(HEADS * NP),
                            sh1.reshape(HEADS * 128),
                            h1p.reshape(HEADS * 2 * NP, HC2))
    h2s, as2, ad2, _, _, sh2 = _tcC(out1.reshape(HEADS, 2, NP, HC2),
                                    den1.reshape(HEADS, NP),
                                    b1.reshape(HEADS, HID_C),
                                    W2.reshape(HEADS, HID_C, OUT_C),
                                    att_src2, att_dst2)
    out2, den2 = _sc_edges2(srcr16, dstr16, as2, ad2, sh2,
                            h2s.reshape(2 * NP, HC2))
    out = _tcE(out2, den2, b2.reshape(1, OUT_C))
    return out[:n]


# speedup vs baseline: 4.4002x; 1.5755x over previous
import jax
import jax.numpy as jnp
from jax import lax
from jax.experimental import pallas as pl
from jax.experimental.pallas import tpu as pltpu
from jax.experimental.pallas import tpu_sc as plsc

N_NODES = 10000
NP = 10240
IN_C = 128
HID_C = 64
OUT_C = 64
HEADS = 8
NEG_SLOPE = 0.2
E_RAW = 320000
E_TOT = E_RAW + N_NODES
EP = 335872
EROWS = EP // 128
ROWS_L1 = EROWS // 16
ROWS_L2 = EROWS // 32
NB = NP // 1024
HC2 = OUT_C // 2

_mesh = plsc.VectorSubcoreMesh(
    core_axis_name="c", subcore_axis_name="s", num_cores=2, num_subcores=16)
_sc_params = pltpu.CompilerParams(use_tc_tiling_on_sc=False,
                                  needs_layout_passes=False)


def _tcA_body(x_ref, w_ref, as_w_ref, ad_w_ref,
              h_ref, s_ref, d_ref, ms_ref, md_ref, sh_ref):
    i = pl.program_id(0)

    @pl.when(i == 0)
    def _init():
        ms_ref[...] = jnp.full((HEADS, 128), -jnp.inf, jnp.float32)
        md_ref[...] = jnp.full((HEADS, 128), -jnp.inf, jnp.float32)

    hblk = jnp.dot(x_ref[...], w_ref[...], preferred_element_type=jnp.float32)
    for h in range(HEADS):
        hh = hblk[:, h * HID_C:(h + 1) * HID_C]
        h_ref[h, 0] = hh[:, :HC2]
        h_ref[h, 1] = hh[:, HC2:]
        s = jnp.sum(hh * as_w_ref[h][None, :], axis=1)
        d = jnp.sum(hh * ad_w_ref[h][None, :], axis=1)
        s_ref[h] = s
        d_ref[h] = d
        ms_ref[h] = jnp.maximum(ms_ref[h], jnp.full((128,), jnp.max(s)))
        md_ref[h] = jnp.maximum(md_ref[h], jnp.full((128,), jnp.max(d)))

    @pl.when(i == NB - 1)
    def _fin():
        t = ms_ref[...] + md_ref[...]
        t = jnp.where(t > 0, t, NEG_SLOPE * t)
        sh_ref[...] = jnp.maximum(t, 0.0)


def _tcA(xp, W1, as_w, ad_w):
    f32 = jnp.float32
    return pl.pallas_call(
        _tcA_body,
        grid=(NB,),
        in_specs=[
            pl.BlockSpec((1024, IN_C), lambda i: (i, 0)),
            pl.BlockSpec((IN_C, HEADS * HID_C), lambda i: (0, 0)),
            pl.BlockSpec((HEADS, HID_C), lambda i: (0, 0)),
            pl.BlockSpec((HEADS, HID_C), lambda i: (0, 0)),
        ],
        out_specs=[
            pl.BlockSpec((HEADS, 2, 1024, HC2), lambda i: (0, 0, i, 0)),
            pl.BlockSpec((HEADS, 1024), lambda i: (0, i)),
            pl.BlockSpec((HEADS, 1024), lambda i: (0, i)),
            pl.BlockSpec((HEADS, 128), lambda i: (0, 0)),
            pl.BlockSpec((HEADS, 128), lambda i: (0, 0)),
            pl.BlockSpec((HEADS, 128), lambda i: (0, 0)),
        ],
        out_shape=[
            jax.ShapeDtypeStruct((HEADS, 2, NP, HC2), f32),
            jax.ShapeDtypeStruct((HEADS, NP), f32),
            jax.ShapeDtypeStruct((HEADS, NP), f32),
            jax.ShapeDtypeStruct((HEADS, 128), f32),
            jax.ShapeDtypeStruct((HEADS, 128), f32),
            jax.ShapeDtypeStruct((HEADS, 128), f32),
        ],
    )(xp, W1, as_w, ad_w)


def _edge_pass(rows, cw, hoff, vm_src, vm_dst, vm_as, vm_ad, vm_sh, hsrc,
               spm_o, spm_d, sets, den_pred):
    iota = lax.iota(jnp.int32, 16)

    def _eix(j, S):
        h_, m_, e_, es_, ix_, gs_, ms_, ds_ = S
        for k in range(8):
            s16 = vm_src[j, pl.ds(k * 16, 16)]
            d16 = vm_dst[j, pl.ds(k * 16, 16)]
            ix_[pl.ds(k * 16, 16)] = s16 + hoff
            sv = plsc.load_gather(vm_as, [s16])
            dv = plsc.load_gather(vm_ad, [d16])
            a = sv + dv
            a = jnp.where(a > 0, a, NEG_SLOPE * a)
            e_[pl.ds(k * 16, 16)] = jnp.exp(a - vm_sh[pl.ds(0, 16)])
        pltpu.async_copy(hsrc.at[ix_], h_, gs_)

    def _half(j, t, n_t, S):
        h_, m_, e_, es_, ix_, gs_, ms_, ds_ = S

        @pl.when(t > 0)
        def _w():
            pltpu.make_async_copy(m_, spm_o.at[vm_dst.at[j]], ms_).wait()

        @pl.when(jnp.logical_and(t > 0, den_pred))
        def _wd():
            pltpu.make_async_copy(es_, spm_d.at[vm_dst.at[j]], ds_).wait()

        pltpu.make_async_copy(hsrc.at[ix_], h_, gs_).wait()

        @plsc.parallel_loop(0, 128, unroll=4)
        def _scale(r):
            ev = plsc.load_gather(e_, [jnp.full((16,), r, jnp.int32)])
            for q in range(cw // 16):
                m_[r, pl.ds(q * 16, 16)] = h_[r, pl.ds(q * 16, 16)] * ev

        pltpu.async_copy(m_, spm_o.at[vm_dst.at[j]], ms_, add=True)

        @pl.when(den_pred)
        def _sd():
            for k in range(8):
                es_[pl.ds(k * 16, 16)] = e_[pl.ds(k * 16, 16)]
            pltpu.async_copy(es_, spm_d.at[vm_dst.at[j]], ds_, add=True)

        @pl.when(t < n_t - 1)
        def _p():
            _eix(j + 2, S)

    S0, S1 = sets
    n_t = rows // 2
    _eix(0, S0)
    _eix(1, S1)

    def _body(t, c):
        _half(2 * t, t, n_t, S0)
        _half(2 * t + 1, t, n_t, S1)
        return c

    lax.fori_loop(0, n_t, _body, 0)
    for S in sets:
        h_, m_, e_, es_, ix_, gs_, ms_, ds_ = S
        pltpu.make_async_copy(m_, spm_o.at[vm_dst.at[0]], ms_).wait()

        @pl.when(den_pred)
        def _wd2():
            pltpu.make_async_copy(es_, spm_d.at[vm_dst.at[0]], ds_).wait()


def _zero_bufs(zb, zd, cq=4):
    def _zrow(r, c):
        for q in range(cq):
            zb[r, pl.ds(q * 16, 16)] = jnp.zeros((16,), jnp.float32)
        return c

    lax.fori_loop(0, 128, _zrow, 0)

    def _zdrow(r, c):
        zd[pl.ds(r * 16, 16)] = jnp.zeros((16,), jnp.float32)
        return c

    lax.fori_loop(0, 40, _zdrow, 0)


def _sc1_body(srcr, dstr, asp, adp, shp, h1f,
              out_o, out_d,
              vm_src, vm_dst, vm_as, vm_ad, vm_sh,
              h0, m0, e0, es0, ix0, h1, m1, e1, es1, ix1,
              zb, zd, spm_o, spm_d, gs0, ms0, ds0, gs1, ms1, ds1):
    core = lax.axis_index("c")
    sub = lax.axis_index("s")
    w0 = sub * 640
    sets = ((h0, m0, e0, es0, ix0, gs0, ms0, ds0),
            (h1, m1, e1, es1, ix1, gs1, ms1, ds1))

    _zero_bufs(zb, zd, cq=HC2 // 16)
    pltpu.sync_copy(srcr.at[sub], vm_src)
    pltpu.sync_copy(dstr.at[sub], vm_dst)

    def _pass(p, c):
        head = core * 4 + p // 2
        den = (p % 2) == 0
        aoff = pl.multiple_of(head * NP, 128)
        hoff = pl.multiple_of((core * 8 + p) * NP, 128)

        @pl.when(den)
        def _cp():
            pltpu.sync_copy(asp.at[pl.ds(aoff, NP)], vm_as)
            pltpu.sync_copy(adp.at[pl.ds(aoff, NP)], vm_ad)
            pltpu.sync_copy(
                shp.at[pl.ds(pl.multiple_of(head * 128, 128), 128)], vm_sh)
            pltpu.sync_copy(zd, spm_d.at[pl.ds(w0, 640)])

        for b in range(5):
            pltpu.sync_copy(zb, spm_o.at[pl.ds(w0 + b * 128, 128)])
        plsc.subcore_barrier()
        _edge_pass(ROWS_L1, HC2, hoff, vm_src, vm_dst, vm_as,
                   vm_ad, vm_sh, h1f, spm_o, spm_d, sets, den)
        plsc.subcore_barrier()
        pltpu.sync_copy(spm_o.at[pl.ds(w0, 640)],
                        out_o.at[core * 8 + p, pl.ds(w0, 640)])

        @pl.when(den)
        def _dd():
            pltpu.sync_copy(
                spm_d.at[pl.ds(w0, 640)],
                out_d.at[pl.ds(pl.multiple_of(aoff + w0, 128), 640)])

        plsc.subcore_barrier()
        return c

    lax.fori_loop(0, 8, _pass, 0)


def _sc_edges1(srcr, dstr, asp, adp, shp, h1f):
    f32 = jnp.float32
    i32 = jnp.int32
    bufset = [
        pltpu.VMEM((128, HC2), f32),
        pltpu.VMEM((128, HC2), f32),
        pltpu.VMEM((128,), f32),
        pltpu.VMEM((128,), f32),
        pltpu.VMEM((128,), i32),
    ]
    fn = pl.kernel(
        _sc1_body,
        out_type=[
            jax.ShapeDtypeStruct((HEADS * 2, NP, HC2), f32),
            jax.ShapeDtypeStruct((HEADS * NP,), f32),
        ],
        mesh=_mesh,
        compiler_params=_sc_params,
        scratch_types=[
            pltpu.VMEM((ROWS_L1, 128), i32),
            pltpu.VMEM((ROWS_L1, 128), i32),
            pltpu.VMEM((NP,), f32),
            pltpu.VMEM((NP,), f32),
            pltpu.VMEM((128,), f32),
        ] + bufset + bufset + [
            pltpu.VMEM((128, HC2), f32),
            pltpu.VMEM((640,), f32),
            pltpu.VMEM_SHARED((NP, HC2), f32),
            pltpu.VMEM_SHARED((NP,), f32),
        ] + [pltpu.SemaphoreType.DMA] * 6,
    )
    return fn(srcr, dstr, asp, adp, shp, h1f)


def _tcC_body(p_ref, d_ref, b1_ref, w2_ref, as2_w_ref, ad2_w_ref,
              h2_ref, s2_ref, d2_ref, ms_ref, md_ref, sh_ref):
    i = pl.program_id(0)

    @pl.when(i == 0)
    def _init():
        ms_ref[...] = jnp.full((128,), -jnp.inf, jnp.float32)
        md_ref[...] = jnp.full((128,), -jnp.inf, jnp.float32)

    acc = jnp.zeros((1024, OUT_C), jnp.float32)
    for h in range(HEADS):
        ph = jnp.concatenate((p_ref[h, 0], p_ref[h, 1]), axis=1)
        v = ph / (d_ref[h][:, None] + 1e-16) + b1_ref[h][None, :]
        v = jnp.where(v > 0, v, jnp.exp(v) - 1.0)
        acc = acc + jnp.dot(v, w2_ref[h], preferred_element_type=jnp.float32)
    h2_ref[0] = acc[:, :32]
    h2_ref[1] = acc[:, 32:]
    s2 = jnp.sum(acc * as2_w_ref[0][None, :], axis=1)
    d2 = jnp.sum(acc * ad2_w_ref[0][None, :], axis=1)
    s2_ref[...] = s2
    d2_ref[...] = d2
    ms_ref[...] = jnp.maximum(ms_ref[...], jnp.full((128,), jnp.max(s2)))
    md_ref[...] = jnp.maximum(md_ref[...], jnp.full((128,), jnp.max(d2)))

    @pl.when(i == NB - 1)
    def _fin():
        t = ms_ref[...] + md_ref[...]
        t = jnp.where(t > 0, t, NEG_SLOPE * t)
        sh_ref[...] = jnp.maximum(t, 0.0)


def _tcC(out1, den1, b1r, w2r, as2_w, ad2_w):
    f32 = jnp.float32
    return pl.pallas_call(
        _tcC_body,
        grid=(NB,),
        in_specs=[
            pl.BlockSpec((HEADS, 2, 1024, HC2), lambda i: (0, 0, i, 0)),
            pl.BlockSpec((HEADS, 1024), lambda i: (0, i)),
            pl.BlockSpec((HEADS, HID_C), lambda i: (0, 0)),
            pl.BlockSpec((HEADS, HID_C, OUT_C), lambda i: (0, 0, 0)),
            pl.BlockSpec((1, OUT_C), lambda i: (0, 0)),
            pl.BlockSpec((1, OUT_C), lambda i: (0, 0)),
        ],
        out_specs=[
            pl.BlockSpec((2, 1024, OUT_C // 2), lambda i: (0, i, 0)),
            pl.BlockSpec((1024,), lambda i: (i,)),
            pl.BlockSpec((1024,), lambda i: (i,)),
            pl.BlockSpec((128,), lambda i: (0,)),
            pl.BlockSpec((128,), lambda i: (0,)),
            pl.BlockSpec((128,), lambda i: (0,)),
        ],
        out_shape=[
            jax.ShapeDtypeStruct((2, NP, OUT_C // 2), f32),
            jax.ShapeDtypeStruct((NP,), f32),
            jax.ShapeDtypeStruct((NP,), f32),
            jax.ShapeDtypeStruct((128,), f32),
            jax.ShapeDtypeStruct((128,), f32),
            jax.ShapeDtypeStruct((128,), f32),
        ],
    )(out1, den1, b1r, w2r, as2_w, ad2_w)


def _sc2_body(srcr, dstr, asp, adp, shp, h2f,
              out_o, out_d,
              vm_src, vm_dst, vm_as, vm_ad, vm_sh,
              h0, m0, e0, es0, ix0, h1, m1, e1, es1, ix1,
              zb, zd, spm_o, spm_d, gs0, ms0, ds0, gs1, ms1, ds1):
    core = lax.axis_index("c")
    sub = lax.axis_index("s")
    w0 = sub * 640
    sets = ((h0, m0, e0, es0, ix0, gs0, ms0, ds0),
            (h1, m1, e1, es1, ix1, gs1, ms1, ds1))

    _zero_bufs(zb, zd, cq=HC2 // 16)
    pltpu.sync_copy(srcr.at[sub], vm_src)
    pltpu.sync_copy(dstr.at[sub], vm_dst)
    pltpu.sync_copy(asp, vm_as)
    pltpu.sync_copy(adp, vm_ad)
    pltpu.sync_copy(shp, vm_sh)
    for b in range(5):
        pltpu.sync_copy(zb, spm_o.at[pl.ds(w0 + b * 128, 128)])
    pltpu.sync_copy(zd, spm_d.at[pl.ds(w0, 640)])
    plsc.subcore_barrier()
    hoff = pl.multiple_of(core * NP, 128)
    _edge_pass(ROWS_L1, HC2, hoff, vm_src, vm_dst, vm_as, vm_ad,
               vm_sh, h2f, spm_o, spm_d, sets, core == 0)
    plsc.subcore_barrier()
    pltpu.sync_copy(spm_o.at[pl.ds(w0, 640)], out_o.at[core, pl.ds(w0, 640)])

    @pl.when(core == 0)
    def _dd():
        pltpu.sync_copy(spm_d.at[pl.ds(w0, 640)], out_d.at[pl.ds(w0, 640)])


def _sc_edges2(srcr, dstr, asp, adp, shp, h2f):
    f32 = jnp.float32
    i32 = jnp.int32
    bufset = [
        pltpu.VMEM((128, HC2), f32),
        pltpu.VMEM((128, HC2), f32),
        pltpu.VMEM((128,), f32),
        pltpu.VMEM((128,), f32),
        pltpu.VMEM((128,), i32),
    ]
    fn = pl.kernel(
        _sc2_body,
        out_type=[
            jax.ShapeDtypeStruct((2, NP, HC2), f32),
            jax.ShapeDtypeStruct((NP,), f32),
        ],
        mesh=_mesh,
        compiler_params=_sc_params,
        scratch_types=[
            pltpu.VMEM((ROWS_L1, 128), i32),
            pltpu.VMEM((ROWS_L1, 128), i32),
            pltpu.VMEM((NP,), f32),
            pltpu.VMEM((NP,), f32),
            pltpu.VMEM((128,), f32),
        ] + bufset + bufset + [
            pltpu.VMEM((128, HC2), f32),
            pltpu.VMEM((640,), f32),
            pltpu.VMEM_SHARED((NP, HC2), f32),
            pltpu.VMEM_SHARED((NP,), f32),
        ] + [pltpu.SemaphoreType.DMA] * 6,
    )
    return fn(srcr, dstr, asp, adp, shp, h2f)


def _tcE_body(p_ref, d_ref, b2_ref, o_ref):
    den = d_ref[...]
    full = jnp.concatenate((p_ref[0], p_ref[1]), axis=1)
    o_ref[...] = full / (den[:, None] + 1e-16) + b2_ref[0][None, :]


def _tcE(out2, den2, b2r):
    return pl.pallas_call(
        _tcE_body,
        grid=(NB,),
        in_specs=[
            pl.BlockSpec((2, 1024, HC2), lambda i: (0, i, 0)),
            pl.BlockSpec((1024,), lambda i: (i,)),
            pl.BlockSpec((1, OUT_C), lambda i: (0, 0)),
        ],
        out_specs=pl.BlockSpec((1024, OUT_C), lambda i: (i, 0)),
        out_shape=jax.ShapeDtypeStruct((NP, OUT_C), jnp.float32),
    )(out2, den2, b2r)


def kernel(x, edge_index, W1, att_src1, att_dst1, b1, W2, att_src2,
           att_dst2, b2):
    n = x.shape[0]
    i32 = jnp.int32
    loop = jnp.arange(n, dtype=i32)
    pad = jnp.full((EP - E_TOT,), NP - 1, i32)
    src = jnp.concatenate([edge_index[0].astype(i32), loop, pad])
    dst = jnp.concatenate([edge_index[1].astype(i32), loop, pad])
    srcr16 = src.reshape(16, ROWS_L1, 128)
    dstr16 = dst.reshape(16, ROWS_L1, 128)
    xp = jnp.pad(x, ((0, NP - n), (0, 0)))

    h1p, asp, adp, _, _, sh1 = _tcA(xp, W1, att_src1, att_dst1)
    out1, den1 = _sc_edges1(srcr16, dstr16, asp.reshape(HEADS * NP),
                            adp.reshape(HEADS * NP),
                            sh1.reshape(HEADS * 128),
                            h1p.reshape(HEADS * 2 * NP, HC2))
    h2s, as2, ad2, _, _, sh2 = _tcC(out1.reshape(HEADS, 2, NP, HC2),
                                    den1.reshape(HEADS, NP),
                                    b1.reshape(HEADS, HID_C),
                                    W2.reshape(HEADS, HID_C, OUT_C),
                                    att_src2, att_dst2)
    out2, den2 = _sc_edges2(srcr16, dstr16, as2, ad2, sh2,
                            h2s.reshape(2 * NP, HC2))
    out = _tcE(out2, den2, b2.reshape(1, OUT_C))
    return out[:n]


# speedup vs baseline: 4.5234x; 1.0280x over previous
import jax
import jax.numpy as jnp
from jax import lax
from jax.experimental import pallas as pl
from jax.experimental.pallas import tpu as pltpu
from jax.experimental.pallas import tpu_sc as plsc

N_NODES = 10000
NP = 10240
IN_C = 128
HID_C = 64
OUT_C = 64
HEADS = 8
NEG_SLOPE = 0.2
E_RAW = 320000
E_TOT = E_RAW + N_NODES
EP = 335872
EROWS = EP // 128
ROWS_L1 = EROWS // 16
ROWS_L2 = EROWS // 32
NB = NP // 1024
HC2 = OUT_C // 2

_mesh = plsc.VectorSubcoreMesh(
    core_axis_name="c", subcore_axis_name="s", num_cores=2, num_subcores=16)
_sc_params = pltpu.CompilerParams(use_tc_tiling_on_sc=False,
                                  needs_layout_passes=False)


def _tcA_body(x_ref, w_ref, as_w_ref, ad_w_ref,
              h_ref, s_ref, d_ref, ms_ref, md_ref, sh_ref):
    i = pl.program_id(0)

    @pl.when(i == 0)
    def _init():
        ms_ref[...] = jnp.full((HEADS, 128), -jnp.inf, jnp.float32)
        md_ref[...] = jnp.full((HEADS, 128), -jnp.inf, jnp.float32)

    hblk = jnp.dot(x_ref[...], w_ref[...], preferred_element_type=jnp.float32)
    for h in range(HEADS):
        hh = hblk[:, h * HID_C:(h + 1) * HID_C]
        h_ref[h, 0] = hh[:, :HC2]
        h_ref[h, 1] = hh[:, HC2:]
        s = jnp.sum(hh * as_w_ref[h][None, :], axis=1)
        d = jnp.sum(hh * ad_w_ref[h][None, :], axis=1)
        s_ref[h] = s
        d_ref[h] = d
        ms_ref[h] = jnp.maximum(ms_ref[h], jnp.full((128,), jnp.max(s)))
        md_ref[h] = jnp.maximum(md_ref[h], jnp.full((128,), jnp.max(d)))

    @pl.when(i == NB - 1)
    def _fin():
        t = ms_ref[...] + md_ref[...]
        t = jnp.where(t > 0, t, NEG_SLOPE * t)
        sh_ref[...] = jnp.maximum(t, 0.0)


def _tcA(xp, W1, as_w, ad_w):
    f32 = jnp.float32
    return pl.pallas_call(
        _tcA_body,
        grid=(NB,),
        in_specs=[
            pl.BlockSpec((1024, IN_C), lambda i: (i, 0)),
            pl.BlockSpec((IN_C, HEADS * HID_C), lambda i: (0, 0)),
            pl.BlockSpec((HEADS, HID_C), lambda i: (0, 0)),
            pl.BlockSpec((HEADS, HID_C), lambda i: (0, 0)),
        ],
        out_specs=[
            pl.BlockSpec((HEADS, 2, 1024, HC2), lambda i: (0, 0, i, 0)),
            pl.BlockSpec((HEADS, 1024), lambda i: (0, i)),
            pl.BlockSpec((HEADS, 1024), lambda i: (0, i)),
            pl.BlockSpec((HEADS, 128), lambda i: (0, 0)),
            pl.BlockSpec((HEADS, 128), lambda i: (0, 0)),
            pl.BlockSpec((HEADS, 128), lambda i: (0, 0)),
        ],
        out_shape=[
            jax.ShapeDtypeStruct((HEADS, 2, NP, HC2), f32),
            jax.ShapeDtypeStruct((HEADS, NP), f32),
            jax.ShapeDtypeStruct((HEADS, NP), f32),
            jax.ShapeDtypeStruct((HEADS, 128), f32),
            jax.ShapeDtypeStruct((HEADS, 128), f32),
            jax.ShapeDtypeStruct((HEADS, 128), f32),
        ],
    )(xp, W1, as_w, ad_w)


def _edge_pass(rows, cw, hoff, vm_src, vm_dst, vm_as, vm_ad, vm_sh, hsrc,
               spm_o, spm_d, sets, den_pred):
    iota = lax.iota(jnp.int32, 16)

    def _eix(j, S):
        h_, m_, e_, es_, ix_, gs_, ms_, ds_ = S

        @plsc.parallel_loop(0, 128, step=16, unroll=4)
        def _ek(k0):
            k = pl.multiple_of(k0, 16)
            s16 = vm_src[j, pl.ds(k, 16)]
            d16 = vm_dst[j, pl.ds(k, 16)]
            ix_[pl.ds(k, 16)] = s16 + hoff
            sv = plsc.load_gather(vm_as, [s16])
            dv = plsc.load_gather(vm_ad, [d16])
            a = sv + dv
            a = jnp.where(a > 0, a, NEG_SLOPE * a)
            e_[pl.ds(k, 16)] = jnp.exp(a - vm_sh[pl.ds(0, 16)])

        pltpu.async_copy(hsrc.at[ix_], h_, gs_)

    def _half(j, t, n_t, S):
        h_, m_, e_, es_, ix_, gs_, ms_, ds_ = S

        @pl.when(t > 0)
        def _w():
            pltpu.make_async_copy(m_, spm_o.at[vm_dst.at[j]], ms_).wait()

        @pl.when(jnp.logical_and(t > 0, den_pred))
        def _wd():
            pltpu.make_async_copy(es_, spm_d.at[vm_dst.at[j]], ds_).wait()

        pltpu.make_async_copy(hsrc.at[ix_], h_, gs_).wait()

        @plsc.parallel_loop(0, 128, unroll=8)
        def _scale(r):
            ev = plsc.load_gather(e_, [jnp.full((16,), r, jnp.int32)])
            for q in range(cw // 16):
                m_[r, pl.ds(q * 16, 16)] = h_[r, pl.ds(q * 16, 16)] * ev

        pltpu.async_copy(m_, spm_o.at[vm_dst.at[j]], ms_, add=True)

        @pl.when(den_pred)
        def _sd():
            for k in range(8):
                es_[pl.ds(k * 16, 16)] = e_[pl.ds(k * 16, 16)]
            pltpu.async_copy(es_, spm_d.at[vm_dst.at[j]], ds_, add=True)

        @pl.when(t < n_t - 1)
        def _p():
            _eix(j + 2, S)

    S0, S1 = sets
    n_t = rows // 2
    _eix(0, S0)
    _eix(1, S1)

    def _body(t, c):
        _half(2 * t, t, n_t, S0)
        _half(2 * t + 1, t, n_t, S1)
        return c

    lax.fori_loop(0, n_t, _body, 0)
    for S in sets:
        h_, m_, e_, es_, ix_, gs_, ms_, ds_ = S
        pltpu.make_async_copy(m_, spm_o.at[vm_dst.at[0]], ms_).wait()

        @pl.when(den_pred)
        def _wd2():
            pltpu.make_async_copy(es_, spm_d.at[vm_dst.at[0]], ds_).wait()


def _zero_bufs(zb, zd, cq=4):
    def _zrow(r, c):
        for q in range(cq):
            zb[r, pl.ds(q * 16, 16)] = jnp.zeros((16,), jnp.float32)
        return c

    lax.fori_loop(0, 128, _zrow, 0)

    def _zdrow(r, c):
        zd[pl.ds(r * 16, 16)] = jnp.zeros((16,), jnp.float32)
        return c

    lax.fori_loop(0, 40, _zdrow, 0)


def _sc1_body(srcr, dstr, asp, adp, shp, h1f,
              out_o, out_d,
              vm_src, vm_dst, vm_as, vm_ad, vm_sh,
              h0, m0, e0, es0, ix0, h1, m1, e1, es1, ix1,
              zb, zd, spm_o, spm_d, gs0, ms0, ds0, gs1, ms1, ds1):
    core = lax.axis_index("c")
    sub = lax.axis_index("s")
    w0 = sub * 640
    sets = ((h0, m0, e0, es0, ix0, gs0, ms0, ds0),
            (h1, m1, e1, es1, ix1, gs1, ms1, ds1))

    _zero_bufs(zb, zd, cq=HC2 // 16)
    pltpu.sync_copy(srcr.at[sub], vm_src)
    pltpu.sync_copy(dstr.at[sub], vm_dst)

    def _pass(p, c):
        head = core * 4 + p // 2
        den = (p % 2) == 0
        aoff = pl.multiple_of(head * NP, 128)
        hoff = pl.multiple_of((core * 8 + p) * NP, 128)

        @pl.when(den)
        def _cp():
            pltpu.sync_copy(asp.at[pl.ds(aoff, NP)], vm_as)
            pltpu.sync_copy(adp.at[pl.ds(aoff, NP)], vm_ad)
            pltpu.sync_copy(
                shp.at[pl.ds(pl.multiple_of(head * 128, 128), 128)], vm_sh)
            pltpu.sync_copy(zd, spm_d.at[pl.ds(w0, 640)])

        for b in range(5):
            pltpu.sync_copy(zb, spm_o.at[pl.ds(w0 + b * 128, 128)])
        plsc.subcore_barrier()
        _edge_pass(ROWS_L1, HC2, hoff, vm_src, vm_dst, vm_as,
                   vm_ad, vm_sh, h1f, spm_o, spm_d, sets, den)
        plsc.subcore_barrier()
        pltpu.sync_copy(spm_o.at[pl.ds(w0, 640)],
                        out_o.at[core * 8 + p, pl.ds(w0, 640)])

        @pl.when(den)
        def _dd():
            pltpu.sync_copy(
                spm_d.at[pl.ds(w0, 640)],
                out_d.at[pl.ds(pl.multiple_of(aoff + w0, 128), 640)])

        plsc.subcore_barrier()
        return c

    lax.fori_loop(0, 8, _pass, 0)


def _sc_edges1(srcr, dstr, asp, adp, shp, h1f):
    f32 = jnp.float32
    i32 = jnp.int32
    bufset = [
        pltpu.VMEM((128, HC2), f32),
        pltpu.VMEM((128, HC2), f32),
        pltpu.VMEM((128,), f32),
        pltpu.VMEM((128,), f32),
        pltpu.VMEM((128,), i32),
    ]
    fn = pl.kernel(
        _sc1_body,
        out_type=[
            jax.ShapeDtypeStruct((HEADS * 2, NP, HC2), f32),
            jax.ShapeDtypeStruct((HEADS * NP,), f32),
        ],
        mesh=_mesh,
        compiler_params=_sc_params,
        scratch_types=[
            pltpu.VMEM((ROWS_L1, 128), i32),
            pltpu.VMEM((ROWS_L1, 128), i32),
            pltpu.VMEM((NP,), f32),
            pltpu.VMEM((NP,), f32),
            pltpu.VMEM((128,), f32),
        ] + bufset + bufset + [
            pltpu.VMEM((128, HC2), f32),
            pltpu.VMEM((640,), f32),
            pltpu.VMEM_SHARED((NP, HC2), f32),
            pltpu.VMEM_SHARED((NP,), f32),
        ] + [pltpu.SemaphoreType.DMA] * 6,
    )
    return fn(srcr, dstr, asp, adp, shp, h1f)


def _tcC_body(p_ref, d_ref, b1_ref, w2_ref, as2_w_ref, ad2_w_ref,
              h2_ref, s2_ref, d2_ref, ms_ref, md_ref, sh_ref):
    i = pl.program_id(0)

    @pl.when(i == 0)
    def _init():
        ms_ref[...] = jnp.full((128,), -jnp.inf, jnp.float32)
        md_ref[...] = jnp.full((128,), -jnp.inf, jnp.float32)

    acc = jnp.zeros((1024, OUT_C), jnp.float32)
    for h in range(HEADS):
        ph = jnp.concatenate((p_ref[h, 0], p_ref[h, 1]), axis=1)
        v = ph / (d_ref[h][:, None] + 1e-16) + b1_ref[h][None, :]
        v = jnp.where(v > 0, v, jnp.exp(v) - 1.0)
        acc = acc + jnp.dot(v, w2_ref[h], preferred_element_type=jnp.float32)
    h2_ref[0] = acc[:, :32]
    h2_ref[1] = acc[:, 32:]
    s2 = jnp.sum(acc * as2_w_ref[0][None, :], axis=1)
    d2 = jnp.sum(acc * ad2_w_ref[0][None, :], axis=1)
    s2_ref[...] = s2
    d2_ref[...] = d2
    ms_ref[...] = jnp.maximum(ms_ref[...], jnp.full((128,), jnp.max(s2)))
    md_ref[...] = jnp.maximum(md_ref[...], jnp.full((128,), jnp.max(d2)))

    @pl.when(i == NB - 1)
    def _fin():
        t = ms_ref[...] + md_ref[...]
        t = jnp.where(t > 0, t, NEG_SLOPE * t)
        sh_ref[...] = jnp.maximum(t, 0.0)


def _tcC(out1, den1, b1r, w2r, as2_w, ad2_w):
    f32 = jnp.float32
    return pl.pallas_call(
        _tcC_body,
        grid=(NB,),
        in_specs=[
            pl.BlockSpec((HEADS, 2, 1024, HC2), lambda i: (0, 0, i, 0)),
            pl.BlockSpec((HEADS, 1024), lambda i: (0, i)),
            pl.BlockSpec((HEADS, HID_C), lambda i: (0, 0)),
            pl.BlockSpec((HEADS, HID_C, OUT_C), lambda i: (0, 0, 0)),
            pl.BlockSpec((1, OUT_C), lambda i: (0, 0)),
            pl.BlockSpec((1, OUT_C), lambda i: (0, 0)),
        ],
        out_specs=[
            pl.BlockSpec((2, 1024, OUT_C // 2), lambda i: (0, i, 0)),
            pl.BlockSpec((1024,), lambda i: (i,)),
            pl.BlockSpec((1024,), lambda i: (i,)),
            pl.BlockSpec((128,), lambda i: (0,)),
            pl.BlockSpec((128,), lambda i: (0,)),
            pl.BlockSpec((128,), lambda i: (0,)),
        ],
        out_shape=[
            jax.ShapeDtypeStruct((2, NP, OUT_C // 2), f32),
            jax.ShapeDtypeStruct((NP,), f32),
            jax.ShapeDtypeStruct((NP,), f32),
            jax.ShapeDtypeStruct((128,), f32),
            jax.ShapeDtypeStruct((128,), f32),
            jax.ShapeDtypeStruct((128,), f32),
        ],
    )(out1, den1, b1r, w2r, as2_w, ad2_w)


def _sc2_body(srcr, dstr, asp, adp, shp, h2f,
              out_o, out_d,
              vm_src, vm_dst, vm_as, vm_ad, vm_sh,
              h0, m0, e0, es0, ix0, h1, m1, e1, es1, ix1,
              zb, zd, spm_o, spm_d, gs0, ms0, ds0, gs1, ms1, ds1):
    core = lax.axis_index("c")
    sub = lax.axis_index("s")
    w0 = sub * 640
    sets = ((h0, m0, e0, es0, ix0, gs0, ms0, ds0),
            (h1, m1, e1, es1, ix1, gs1, ms1, ds1))

    _zero_bufs(zb, zd, cq=HC2 // 16)
    pltpu.sync_copy(srcr.at[sub], vm_src)
    pltpu.sync_copy(dstr.at[sub], vm_dst)
    pltpu.sync_copy(asp, vm_as)
    pltpu.sync_copy(adp, vm_ad)
    pltpu.sync_copy(shp, vm_sh)
    for b in range(5):
        pltpu.sync_copy(zb, spm_o.at[pl.ds(w0 + b * 128, 128)])
    pltpu.sync_copy(zd, spm_d.at[pl.ds(w0, 640)])
    plsc.subcore_barrier()
    hoff = pl.multiple_of(core * NP, 128)
    _edge_pass(ROWS_L1, HC2, hoff, vm_src, vm_dst, vm_as, vm_ad,
               vm_sh, h2f, spm_o, spm_d, sets, core == 0)
    plsc.subcore_barrier()
    pltpu.sync_copy(spm_o.at[pl.ds(w0, 640)], out_o.at[core, pl.ds(w0, 640)])

    @pl.when(core == 0)
    def _dd():
        pltpu.sync_copy(spm_d.at[pl.ds(w0, 640)], out_d.at[pl.ds(w0, 640)])


def _sc_edges2(srcr, dstr, asp, adp, shp, h2f):
    f32 = jnp.float32
    i32 = jnp.int32
    bufset = [
        pltpu.VMEM((128, HC2), f32),
        pltpu.VMEM((128, HC2), f32),
        pltpu.VMEM((128,), f32),
        pltpu.VMEM((128,), f32),
        pltpu.VMEM((128,), i32),
    ]
    fn = pl.kernel(
        _sc2_body,
        out_type=[
            jax.ShapeDtypeStruct((2, NP, HC2), f32),
            jax.ShapeDtypeStruct((NP,), f32),
        ],
        mesh=_mesh,
        compiler_params=_sc_params,
        scratch_types=[
            pltpu.VMEM((ROWS_L1, 128), i32),
            pltpu.VMEM((ROWS_L1, 128), i32),
            pltpu.VMEM((NP,), f32),
            pltpu.VMEM((NP,), f32),
            pltpu.VMEM((128,), f32),
        ] + bufset + bufset + [
            pltpu.VMEM((128, HC2), f32),
            pltpu.VMEM((640,), f32),
            pltpu.VMEM_SHARED((NP, HC2), f32),
            pltpu.VMEM_SHARED((NP,), f32),
        ] + [pltpu.SemaphoreType.DMA] * 6,
    )
    return fn(srcr, dstr, asp, adp, shp, h2f)


def _tcE_body(p_ref, d_ref, b2_ref, o_ref):
    den = d_ref[...]
    full = jnp.concatenate((p_ref[0], p_ref[1]), axis=1)
    o_ref[...] = full / (den[:, None] + 1e-16) + b2_ref[0][None, :]


def _tcE(out2, den2, b2r):
    return pl.pallas_call(
        _tcE_body,
        grid=(NB,),
        in_specs=[
            pl.BlockSpec((2, 1024, HC2), lambda i: (0, i, 0)),
            pl.BlockSpec((1024,), lambda i: (i,)),
            pl.BlockSpec((1, OUT_C), lambda i: (0, 0)),
        ],
        out_specs=pl.BlockSpec((1024, OUT_C), lambda i: (i, 0)),
        out_shape=jax.ShapeDtypeStruct((NP, OUT_C), jnp.float32),
    )(out2, den2, b2r)


def kernel(x, edge_index, W1, att_src1, att_dst1, b1, W2, att_src2,
           att_dst2, b2):
    n = x.shape[0]
    i32 = jnp.int32
    loop = jnp.arange(n, dtype=i32)
    pad = jnp.full((EP - E_TOT,), NP - 1, i32)
    src = jnp.concatenate([edge_index[0].astype(i32), loop, pad])
    dst = jnp.concatenate([edge_index[1].astype(i32), loop, pad])
    srcr16 = src.reshape(16, ROWS_L1, 128)
    dstr16 = dst.reshape(16, ROWS_L1, 128)
    xp = jnp.pad(x, ((0, NP - n), (0, 0)))

    h1p, asp, adp, _, _, sh1 = _tcA(xp, W1, att_src1, att_dst1)
    out1, den1 = _sc_edges1(srcr16, dstr16, asp.reshape(HEADS * NP),
                            adp.reshape(HEADS * NP),
                            sh1.reshape(HEADS * 128),
                            h1p.reshape(HEADS * 2 * NP, HC2))
    h2s, as2, ad2, _, _, sh2 = _tcC(out1.reshape(HEADS, 2, NP, HC2),
                                    den1.reshape(HEADS, NP),
                                    b1.reshape(HEADS, HID_C),
                                    W2.reshape(HEADS, HID_C, OUT_C),
                                    att_src2, att_dst2)
    out2, den2 = _sc_edges2(srcr16, dstr16, as2, ad2, sh2,
                            h2s.reshape(2 * NP, HC2))
    out = _tcE(out2, den2, b2.reshape(1, OUT_C))
    return out[:n]


# speedup vs baseline: 5.0057x; 1.1066x over previous
import jax
import jax.numpy as jnp
from jax import lax
from jax.experimental import pallas as pl
from jax.experimental.pallas import tpu as pltpu
from jax.experimental.pallas import tpu_sc as plsc

N_NODES = 10000
NP = 10240
IN_C = 128
HID_C = 64
OUT_C = 64
HEADS = 8
NEG_SLOPE = 0.2
E_RAW = 320000
E_TOT = E_RAW + N_NODES
EP = 335872
EROWS = EP // 128
ROWS_L1 = EROWS // 16
ROWS_L2 = EROWS // 32
NB = NP // 1024
HC2 = OUT_C // 2

_mesh = plsc.VectorSubcoreMesh(
    core_axis_name="c", subcore_axis_name="s", num_cores=2, num_subcores=16)
_sc_params = pltpu.CompilerParams(use_tc_tiling_on_sc=False,
                                  needs_layout_passes=False)


def _tcA_body(x_ref, w_ref, as_w_ref, ad_w_ref,
              h_ref, s_ref, d_ref, ms_ref, md_ref, sh_ref):
    i = pl.program_id(0)

    @pl.when(i == 0)
    def _init():
        ms_ref[...] = jnp.full((HEADS, 128), -jnp.inf, jnp.float32)
        md_ref[...] = jnp.full((HEADS, 128), -jnp.inf, jnp.float32)

    hblk = jnp.dot(x_ref[...], w_ref[...], preferred_element_type=jnp.float32)
    for h in range(HEADS):
        hh = hblk[:, h * HID_C:(h + 1) * HID_C]
        h_ref[h, 0] = hh[:, :HC2]
        h_ref[h, 1] = hh[:, HC2:]
        s = jnp.sum(hh * as_w_ref[h][None, :], axis=1)
        d = jnp.sum(hh * ad_w_ref[h][None, :], axis=1)
        s_ref[h] = s
        d_ref[h] = d
        ms_ref[h] = jnp.maximum(ms_ref[h], jnp.full((128,), jnp.max(s)))
        md_ref[h] = jnp.maximum(md_ref[h], jnp.full((128,), jnp.max(d)))

    @pl.when(i == NB - 1)
    def _fin():
        t = ms_ref[...] + md_ref[...]
        t = jnp.where(t > 0, t, NEG_SLOPE * t)
        sh_ref[...] = jnp.maximum(t, 0.0)


def _tcA(xp, W1, as_w, ad_w):
    f32 = jnp.float32
    return pl.pallas_call(
        _tcA_body,
        grid=(NB,),
        in_specs=[
            pl.BlockSpec((1024, IN_C), lambda i: (i, 0)),
            pl.BlockSpec((IN_C, HEADS * HID_C), lambda i: (0, 0)),
            pl.BlockSpec((HEADS, HID_C), lambda i: (0, 0)),
            pl.BlockSpec((HEADS, HID_C), lambda i: (0, 0)),
        ],
        out_specs=[
            pl.BlockSpec((HEADS, 2, 1024, HC2), lambda i: (0, 0, i, 0)),
            pl.BlockSpec((HEADS, 1024), lambda i: (0, i)),
            pl.BlockSpec((HEADS, 1024), lambda i: (0, i)),
            pl.BlockSpec((HEADS, 128), lambda i: (0, 0)),
            pl.BlockSpec((HEADS, 128), lambda i: (0, 0)),
            pl.BlockSpec((HEADS, 128), lambda i: (0, 0)),
        ],
        out_shape=[
            jax.ShapeDtypeStruct((HEADS, 2, NP, HC2), f32),
            jax.ShapeDtypeStruct((HEADS, NP), f32),
            jax.ShapeDtypeStruct((HEADS, NP), f32),
            jax.ShapeDtypeStruct((HEADS, 128), f32),
            jax.ShapeDtypeStruct((HEADS, 128), f32),
            jax.ShapeDtypeStruct((HEADS, 128), f32),
        ],
    )(xp, W1, as_w, ad_w)


def _edge_pass(rows, cw, hoff, vm_src, vm_dst, vm_as, vm_ad, vm_sh, hsrc,
               spm_o, spm_d, sets, den_pred):
    iota = lax.iota(jnp.int32, 16)

    def _eix(j, S):
        h_, m_, e_, es_, ix_, gs_, ms_, ds_ = S

        @plsc.parallel_loop(0, 128, step=16, unroll=4)
        def _ek(k0):
            k = pl.multiple_of(k0, 16)
            s16 = vm_src[j, pl.ds(k, 16)]
            d16 = vm_dst[j, pl.ds(k, 16)]
            ix_[pl.ds(k, 16)] = s16 + hoff
            sv = plsc.load_gather(vm_as, [s16])
            dv = plsc.load_gather(vm_ad, [d16])
            a = sv + dv
            a = jnp.where(a > 0, a, NEG_SLOPE * a)
            e_[pl.ds(k, 16)] = jnp.exp(a - vm_sh[pl.ds(0, 16)])

        pltpu.async_copy(hsrc.at[ix_], h_, gs_)

    def _half(j, t, n_t, S):
        h_, m_, e_, es_, ix_, gs_, ms_, ds_ = S

        @pl.when(t > 0)
        def _w():
            pltpu.make_async_copy(m_, spm_o.at[vm_dst.at[j]], ms_).wait()

        @pl.when(jnp.logical_and(t > 0, den_pred))
        def _wd():
            pltpu.make_async_copy(es_, spm_d.at[vm_dst.at[j]], ds_).wait()

        pltpu.make_async_copy(hsrc.at[ix_], h_, gs_).wait()

        @plsc.parallel_loop(0, 128, unroll=8)
        def _scale(r):
            ev = plsc.load_gather(e_, [jnp.full((16,), r, jnp.int32)])
            for q in range(cw // 16):
                m_[r, pl.ds(q * 16, 16)] = h_[r, pl.ds(q * 16, 16)] * ev

        pltpu.async_copy(m_, spm_o.at[vm_dst.at[j]], ms_, add=True)

        @pl.when(den_pred)
        def _sd():
            for k in range(8):
                es_[pl.ds(k * 16, 16)] = e_[pl.ds(k * 16, 16)]
            pltpu.async_copy(es_, spm_d.at[vm_dst.at[j]], ds_, add=True)

        @pl.when(t < n_t - 1)
        def _p():
            _eix(j + len(sets), S)

    ns = len(sets)
    n_t = rows // ns
    for i, S in enumerate(sets):
        _eix(i, S)

    def _body(t, c):
        for i, S in enumerate(sets):
            _half(ns * t + i, t, n_t, S)
        return c

    lax.fori_loop(0, n_t, _body, 0)
    for S in sets:
        h_, m_, e_, es_, ix_, gs_, ms_, ds_ = S
        pltpu.make_async_copy(m_, spm_o.at[vm_dst.at[0]], ms_).wait()

        @pl.when(den_pred)
        def _wd2():
            pltpu.make_async_copy(es_, spm_d.at[vm_dst.at[0]], ds_).wait()


def _zero_bufs(zb, zd, cq=4):
    def _zrow(r, c):
        for q in range(cq):
            zb[r, pl.ds(q * 16, 16)] = jnp.zeros((16,), jnp.float32)
        return c

    lax.fori_loop(0, 128, _zrow, 0)

    def _zdrow(r, c):
        zd[pl.ds(r * 16, 16)] = jnp.zeros((16,), jnp.float32)
        return c

    lax.fori_loop(0, 40, _zdrow, 0)


NSETS = 4


def _unpack_scratch(scratch):
    vm_src, vm_dst, vm_as, vm_ad, vm_sh = scratch[:5]
    bufs = scratch[5:5 + 5 * NSETS]
    zb, zd, spm_o, spm_d = scratch[5 + 5 * NSETS:9 + 5 * NSETS]
    sems = scratch[9 + 5 * NSETS:]
    sets = tuple(tuple(bufs[5 * i:5 * i + 5]) + tuple(sems[3 * i:3 * i + 3])
                 for i in range(NSETS))
    return vm_src, vm_dst, vm_as, vm_ad, vm_sh, zb, zd, spm_o, spm_d, sets


def _sc1_body(srcr, dstr, asp, adp, shp, h1f, out_o, out_d, *scratch):
    core = lax.axis_index("c")
    sub = lax.axis_index("s")
    w0 = sub * 640
    (vm_src, vm_dst, vm_as, vm_ad, vm_sh, zb, zd, spm_o, spm_d,
     sets) = _unpack_scratch(scratch)

    _zero_bufs(zb, zd, cq=HC2 // 16)
    pltpu.sync_copy(srcr.at[sub], vm_src)
    pltpu.sync_copy(dstr.at[sub], vm_dst)

    def _pass(p, c):
        head = core * 4 + p // 2
        den = (p % 2) == 0
        aoff = pl.multiple_of(head * NP, 128)
        hoff = pl.multiple_of((core * 8 + p) * NP, 128)

        @pl.when(den)
        def _cp():
            pltpu.sync_copy(asp.at[pl.ds(aoff, NP)], vm_as)
            pltpu.sync_copy(adp.at[pl.ds(aoff, NP)], vm_ad)
            pltpu.sync_copy(
                shp.at[pl.ds(pl.multiple_of(head * 128, 128), 128)], vm_sh)
            pltpu.sync_copy(zd, spm_d.at[pl.ds(w0, 640)])

        for b in range(5):
            pltpu.sync_copy(zb, spm_o.at[pl.ds(w0 + b * 128, 128)])
        plsc.subcore_barrier()
        _edge_pass(ROWS_L1, HC2, hoff, vm_src, vm_dst, vm_as,
                   vm_ad, vm_sh, h1f, spm_o, spm_d, sets, den)
        plsc.subcore_barrier()
        pltpu.sync_copy(spm_o.at[pl.ds(w0, 640)],
                        out_o.at[core * 8 + p, pl.ds(w0, 640)])

        @pl.when(den)
        def _dd():
            pltpu.sync_copy(
                spm_d.at[pl.ds(w0, 640)],
                out_d.at[pl.ds(pl.multiple_of(aoff + w0, 128), 640)])

        plsc.subcore_barrier()
        return c

    lax.fori_loop(0, 8, _pass, 0)


def _sc_edges1(srcr, dstr, asp, adp, shp, h1f):
    f32 = jnp.float32
    i32 = jnp.int32
    bufset = [
        pltpu.VMEM((128, HC2), f32),
        pltpu.VMEM((128, HC2), f32),
        pltpu.VMEM((128,), f32),
        pltpu.VMEM((128,), f32),
        pltpu.VMEM((128,), i32),
    ]
    fn = pl.kernel(
        _sc1_body,
        out_type=[
            jax.ShapeDtypeStruct((HEADS * 2, NP, HC2), f32),
            jax.ShapeDtypeStruct((HEADS * NP,), f32),
        ],
        mesh=_mesh,
        compiler_params=_sc_params,
        scratch_types=[
            pltpu.VMEM((ROWS_L1, 128), i32),
            pltpu.VMEM((ROWS_L1, 128), i32),
            pltpu.VMEM((NP,), f32),
            pltpu.VMEM((NP,), f32),
            pltpu.VMEM((128,), f32),
        ] + bufset * NSETS + [
            pltpu.VMEM((128, HC2), f32),
            pltpu.VMEM((640,), f32),
            pltpu.VMEM_SHARED((NP, HC2), f32),
            pltpu.VMEM_SHARED((NP,), f32),
        ] + [pltpu.SemaphoreType.DMA] * (3 * NSETS),
    )
    return fn(srcr, dstr, asp, adp, shp, h1f)


def _tcC_body(p_ref, d_ref, b1_ref, w2_ref, as2_w_ref, ad2_w_ref,
              h2_ref, s2_ref, d2_ref, ms_ref, md_ref, sh_ref):
    i = pl.program_id(0)

    @pl.when(i == 0)
    def _init():
        ms_ref[...] = jnp.full((128,), -jnp.inf, jnp.float32)
        md_ref[...] = jnp.full((128,), -jnp.inf, jnp.float32)

    acc = jnp.zeros((1024, OUT_C), jnp.float32)
    for h in range(HEADS):
        ph = jnp.concatenate((p_ref[h, 0], p_ref[h, 1]), axis=1)
        v = ph / (d_ref[h][:, None] + 1e-16) + b1_ref[h][None, :]
        v = jnp.where(v > 0, v, jnp.exp(v) - 1.0)
        acc = acc + jnp.dot(v, w2_ref[h], preferred_element_type=jnp.float32)
    h2_ref[0] = acc[:, :32]
    h2_ref[1] = acc[:, 32:]
    s2 = jnp.sum(acc * as2_w_ref[0][None, :], axis=1)
    d2 = jnp.sum(acc * ad2_w_ref[0][None, :], axis=1)
    s2_ref[...] = s2
    d2_ref[...] = d2
    ms_ref[...] = jnp.maximum(ms_ref[...], jnp.full((128,), jnp.max(s2)))
    md_ref[...] = jnp.maximum(md_ref[...], jnp.full((128,), jnp.max(d2)))

    @pl.when(i == NB - 1)
    def _fin():
        t = ms_ref[...] + md_ref[...]
        t = jnp.where(t > 0, t, NEG_SLOPE * t)
        sh_ref[...] = jnp.maximum(t, 0.0)


def _tcC(out1, den1, b1r, w2r, as2_w, ad2_w):
    f32 = jnp.float32
    return pl.pallas_call(
        _tcC_body,
        grid=(NB,),
        in_specs=[
            pl.BlockSpec((HEADS, 2, 1024, HC2), lambda i: (0, 0, i, 0)),
            pl.BlockSpec((HEADS, 1024), lambda i: (0, i)),
            pl.BlockSpec((HEADS, HID_C), lambda i: (0, 0)),
            pl.BlockSpec((HEADS, HID_C, OUT_C), lambda i: (0, 0, 0)),
            pl.BlockSpec((1, OUT_C), lambda i: (0, 0)),
            pl.BlockSpec((1, OUT_C), lambda i: (0, 0)),
        ],
        out_specs=[
            pl.BlockSpec((2, 1024, OUT_C // 2), lambda i: (0, i, 0)),
            pl.BlockSpec((1024,), lambda i: (i,)),
            pl.BlockSpec((1024,), lambda i: (i,)),
            pl.BlockSpec((128,), lambda i: (0,)),
            pl.BlockSpec((128,), lambda i: (0,)),
            pl.BlockSpec((128,), lambda i: (0,)),
        ],
        out_shape=[
            jax.ShapeDtypeStruct((2, NP, OUT_C // 2), f32),
            jax.ShapeDtypeStruct((NP,), f32),
            jax.ShapeDtypeStruct((NP,), f32),
            jax.ShapeDtypeStruct((128,), f32),
            jax.ShapeDtypeStruct((128,), f32),
            jax.ShapeDtypeStruct((128,), f32),
        ],
    )(out1, den1, b1r, w2r, as2_w, ad2_w)


def _sc2_body(srcr, dstr, asp, adp, shp, h2f, out_o, out_d, *scratch):
    core = lax.axis_index("c")
    sub = lax.axis_index("s")
    w0 = sub * 640
    (vm_src, vm_dst, vm_as, vm_ad, vm_sh, zb, zd, spm_o, spm_d,
     sets) = _unpack_scratch(scratch)

    _zero_bufs(zb, zd, cq=HC2 // 16)
    pltpu.sync_copy(srcr.at[sub], vm_src)
    pltpu.sync_copy(dstr.at[sub], vm_dst)
    pltpu.sync_copy(asp, vm_as)
    pltpu.sync_copy(adp, vm_ad)
    pltpu.sync_copy(shp, vm_sh)
    for b in range(5):
        pltpu.sync_copy(zb, spm_o.at[pl.ds(w0 + b * 128, 128)])
    pltpu.sync_copy(zd, spm_d.at[pl.ds(w0, 640)])
    plsc.subcore_barrier()
    hoff = pl.multiple_of(core * NP, 128)
    _edge_pass(ROWS_L1, HC2, hoff, vm_src, vm_dst, vm_as, vm_ad,
               vm_sh, h2f, spm_o, spm_d, sets, core == 0)
    plsc.subcore_barrier()
    pltpu.sync_copy(spm_o.at[pl.ds(w0, 640)], out_o.at[core, pl.ds(w0, 640)])

    @pl.when(core == 0)
    def _dd():
        pltpu.sync_copy(spm_d.at[pl.ds(w0, 640)], out_d.at[pl.ds(w0, 640)])


def _sc_edges2(srcr, dstr, asp, adp, shp, h2f):
    f32 = jnp.float32
    i32 = jnp.int32
    bufset = [
        pltpu.VMEM((128, HC2), f32),
        pltpu.VMEM((128, HC2), f32),
        pltpu.VMEM((128,), f32),
        pltpu.VMEM((128,), f32),
        pltpu.VMEM((128,), i32),
    ]
    fn = pl.kernel(
        _sc2_body,
        out_type=[
            jax.ShapeDtypeStruct((2, NP, HC2), f32),
            jax.ShapeDtypeStruct((NP,), f32),
        ],
        mesh=_mesh,
        compiler_params=_sc_params,
        scratch_types=[
            pltpu.VMEM((ROWS_L1, 128), i32),
            pltpu.VMEM((ROWS_L1, 128), i32),
            pltpu.VMEM((NP,), f32),
            pltpu.VMEM((NP,), f32),
            pltpu.VMEM((128,), f32),
        ] + bufset * NSETS + [
            pltpu.VMEM((128, HC2), f32),
            pltpu.VMEM((640,), f32),
            pltpu.VMEM_SHARED((NP, HC2), f32),
            pltpu.VMEM_SHARED((NP,), f32),
        ] + [pltpu.SemaphoreType.DMA] * (3 * NSETS),
    )
    return fn(srcr, dstr, asp, adp, shp, h2f)


def _tcE_body(p_ref, d_ref, b2_ref, o_ref):
    den = d_ref[...]
    full = jnp.concatenate((p_ref[0], p_ref[1]), axis=1)
    o_ref[...] = full / (den[:, None] + 1e-16) + b2_ref[0][None, :]


def _tcE(out2, den2, b2r):
    return pl.pallas_call(
        _tcE_body,
        grid=(NB,),
        in_specs=[
            pl.BlockSpec((2, 1024, HC2), lambda i: (0, i, 0)),
            pl.BlockSpec((1024,), lambda i: (i,)),
            pl.BlockSpec((1, OUT_C), lambda i: (0, 0)),
        ],
        out_specs=pl.BlockSpec((1024, OUT_C), lambda i: (i, 0)),
        out_shape=jax.ShapeDtypeStruct((NP, OUT_C), jnp.float32),
    )(out2, den2, b2r)


def kernel(x, edge_index, W1, att_src1, att_dst1, b1, W2, att_src2,
           att_dst2, b2):
    n = x.shape[0]
    i32 = jnp.int32
    loop = jnp.arange(n, dtype=i32)
    pad = jnp.full((EP - E_TOT,), NP - 1, i32)
    src = jnp.concatenate([edge_index[0].astype(i32), loop, pad])
    dst = jnp.concatenate([edge_index[1].astype(i32), loop, pad])
    srcr16 = src.reshape(16, ROWS_L1, 128)
    dstr16 = dst.reshape(16, ROWS_L1, 128)
    xp = jnp.pad(x, ((0, NP - n), (0, 0)))

    h1p, asp, adp, _, _, sh1 = _tcA(xp, W1, att_src1, att_dst1)
    out1, den1 = _sc_edges1(srcr16, dstr16, asp.reshape(HEADS * NP),
                            adp.reshape(HEADS * NP),
                            sh1.reshape(HEADS * 128),
                            h1p.reshape(HEADS * 2 * NP, HC2))
    h2s, as2, ad2, _, _, sh2 = _tcC(out1.reshape(HEADS, 2, NP, HC2),
                                    den1.reshape(HEADS, NP),
                                    b1.reshape(HEADS, HID_C),
                                    W2.reshape(HEADS, HID_C, OUT_C),
                                    att_src2, att_dst2)
    out2, den2 = _sc_edges2(srcr16, dstr16, as2, ad2, sh2,
                            h2s.reshape(2 * NP, HC2))
    out = _tcE(out2, den2, b2.reshape(1, OUT_C))
    return out[:n]
